# baseline (speedup 1.0000x reference)
import jax
import jax.numpy as jnp
from jax.experimental import pallas as pl
from jax.experimental.pallas import tpu as pltpu

_KCODES = 512
_EMB = 32
_BETA = 0.25
_HB = 8
_WB = 256


def _vq_body(zc_ref, z2_ref, en_ref, e_ref, e2t_ref, oh_ref, zq_ref, idx_ref,
             esum_ref, zsum_ref, scal_ref):
    en = en_ref[...]
    e = e_ref[...]
    e2t = e2t_ref[...]
    esum_acc = jnp.zeros((1, _KCODES), jnp.float32)
    ones_wb = jnp.ones((1, _WB), jnp.float32)
    zsum_acc = jnp.zeros((_EMB, _WB), jnp.float32)
    dminsum = jnp.float32(0.0)
    z2sum = jnp.float32(0.0)
    for g in range(_HB):
        zg = zc_ref[0, :, g, :]
        z2g = z2_ref[0, g, :]
        m2t = jax.lax.dot_general(en, zg, (((1,), (0,)), ((), ())),
                                  preferred_element_type=jnp.float32)
        dt = (z2g[None, :] + e2t) + m2t
        dmint = jnp.min(dt, axis=0, keepdims=True)
        citer = jax.lax.broadcasted_iota(jnp.int32, dt.shape, 0)
        idxt = jnp.min(jnp.where(dt == dmint, citer, jnp.int32(_KCODES)),
                       axis=0)
        idx_ref[0, g, :] = idxt
        lane = jax.lax.broadcasted_iota(jnp.int32, (_WB, _KCODES), 1)
        oh = (lane == idxt[:, None]).astype(jnp.float32)
        oh_ref[0, g, :, :] = oh
        zqt = jax.lax.dot_general(e, oh, (((0,), (1,)), ((), ())),
                                  preferred_element_type=jnp.float32)
        zq_ref[0, :, g, :] = zqt
        esum_acc += jax.lax.dot_general(
            ones_wb, oh, (((1,), (0,)), ((), ())),
            preferred_element_type=jnp.float32)
        zsum_acc += zg
        dminsum += jnp.sum(dmint)
        z2sum += jnp.sum(z2g)
    esum_ref[0] = esum_acc
    zsum_ref[0] = zsum_acc
    scal_ref[0] = (jnp.pad(dminsum.reshape(1, 1), ((0, 0), (0, 127)))
                   + jnp.pad(z2sum.reshape(1, 1), ((0, 0), (1, 126))))


def kernel(z, emb_weight):
    b, c, h, w, dd = z.shape
    n = b * h * w * dd
    wd = w * dd
    zc = z.reshape(b, c, h, wd)
    z2 = (z ** 2).sum(axis=1).reshape(b, h, wd)
    en = jnp.float32(-2.0) * emb_weight
    e2t = (emb_weight ** 2).sum(axis=1)[:, None]

    gh, gw = h // _HB, wd // _WB
    nstep = b * gh * gw
    grid = (b, gh, gw)

    def _pidx(i, j, k):
        return (i * gh + j) * gw + k

    oh, zqt, idx3, esum_p, zsum_p, scal_p = pl.pallas_call(
        _vq_body,
        grid=grid,
        in_specs=[
            pl.BlockSpec((1, c, _HB, _WB), lambda i, j, k: (i, 0, j, k)),
            pl.BlockSpec((1, _HB, _WB), lambda i, j, k: (i, j, k)),
            pl.BlockSpec((_KCODES, _EMB), lambda i, j, k: (0, 0)),
            pl.BlockSpec((_KCODES, _EMB), lambda i, j, k: (0, 0)),
            pl.BlockSpec((_KCODES, 1), lambda i, j, k: (0, 0)),
        ],
        out_specs=[
            pl.BlockSpec((1, _HB, _WB, _KCODES), lambda i, j, k: (i, j, k, 0)),
            pl.BlockSpec((1, c, _HB, _WB), lambda i, j, k: (i, 0, j, k)),
            pl.BlockSpec((1, _HB, _WB), lambda i, j, k: (i, j, k)),
            pl.BlockSpec((1, 1, _KCODES), lambda i, j, k: (_pidx(i, j, k), 0, 0)),
            pl.BlockSpec((1, _EMB, _WB), lambda i, j, k: (_pidx(i, j, k), 0, 0)),
            pl.BlockSpec((1, 1, 128), lambda i, j, k: (_pidx(i, j, k), 0, 0)),
        ],
        out_shape=[
            jax.ShapeDtypeStruct((b, h, wd, _KCODES), jnp.float32),
            jax.ShapeDtypeStruct((b, c, h, wd), jnp.float32),
            jax.ShapeDtypeStruct((b, h, wd), jnp.int32),
            jax.ShapeDtypeStruct((nstep, 1, _KCODES), jnp.float32),
            jax.ShapeDtypeStruct((nstep, _EMB, _WB), jnp.float32),
            jax.ShapeDtypeStruct((nstep, 1, 128), jnp.float32),
        ],
        compiler_params=pltpu.CompilerParams(
            dimension_semantics=("parallel", "parallel", "parallel")),
    )(zc, z2, en, emb_weight, e2t)

    min_encodings = oh.reshape(n, _KCODES)
    min_encoding_indices = idx3.reshape(n, 1)
    z_q_st = zqt.reshape(b, c, h, w, dd)
    dminsum = jnp.sum(scal_p[:, 0, 0])
    z2sum = jnp.sum(scal_p[:, 0, 1])
    esum = jnp.sum(esum_p[:, 0, :], axis=0)
    zsum = jnp.sum(zsum_p, axis=(0, 2))
    dsum = (jnp.float32(_KCODES) * z2sum + jnp.float32(n) * jnp.sum(e2t)
            - 2.0 * jnp.sum(zsum * jnp.sum(emb_weight, axis=0)))
    mean_distance = dsum / jnp.float32(n * _KCODES)
    loss = (1.0 + _BETA) * dminsum / jnp.float32(n * c)
    e_mean = esum / jnp.float32(n)
    perplexity = jnp.exp(-jnp.sum(e_mean * jnp.log(e_mean + 1e-10)))
    return (z_q_st, loss, perplexity, min_encodings, min_encoding_indices,
            mean_distance)


# speedup vs baseline: 1.1526x; 1.1526x over previous
import jax
import jax.numpy as jnp
from jax.experimental import pallas as pl
from jax.experimental.pallas import tpu as pltpu

_KCODES = 512
_EMB = 32
_BETA = 0.25
_HB = 8
_WB = 512


def _vq_body(zc_ref, z2_ref, en_ref, e_ref, e2t_ref, oh_ref, zq_ref, idx_ref,
             esum_ref, zsum_ref, scal_ref):
    en = en_ref[...]
    e = e_ref[...]
    e2t = e2t_ref[...]
    esum_acc = jnp.zeros((1, _KCODES), jnp.float32)
    zsum_acc = jnp.zeros((_EMB, _WB), jnp.float32)
    dminsum = jnp.float32(0.0)
    z2sum = jnp.float32(0.0)
    for g in range(_HB):
        zg = zc_ref[0, :, g, :]
        z2g = z2_ref[0, g, :]
        m2t = jax.lax.dot_general(en, zg, (((1,), (0,)), ((), ())),
                                  preferred_element_type=jnp.float32)
        dt = (z2g[None, :] + e2t) + m2t
        dmint = jnp.min(dt, axis=0, keepdims=True)
        citer = jax.lax.broadcasted_iota(jnp.int32, dt.shape, 0)
        idxt = jnp.min(jnp.where(dt == dmint, citer, jnp.int32(_KCODES)),
                       axis=0)
        idx_ref[0, g, :] = idxt
        lane = jax.lax.broadcasted_iota(jnp.int32, (_WB, _KCODES), 1)
        oh = (lane == idxt[:, None]).astype(jnp.float32)
        oh_ref[0, g, :, :] = oh
        zqt = jax.lax.dot_general(e, oh, (((0,), (1,)), ((), ())),
                                  preferred_element_type=jnp.float32)
        zq_ref[0, :, g, :] = zqt
        esum_acc += jnp.sum(oh, axis=0)[None, :]
        zsum_acc += zg
        dminsum += jnp.sum(dmint)
        z2sum += jnp.sum(z2g)
    esum_ref[0] = esum_acc
    zsum_ref[0] = zsum_acc
    scal_ref[0] = (jnp.pad(dminsum.reshape(1, 1), ((0, 0), (0, 127)))
                   + jnp.pad(z2sum.reshape(1, 1), ((0, 0), (1, 126))))


def kernel(z, emb_weight):
    b, c, h, w, dd = z.shape
    n = b * h * w * dd
    wd = w * dd
    zc = z.reshape(b, c, h, wd)
    z2 = (z ** 2).sum(axis=1).reshape(b, h, wd)
    en = jnp.float32(-2.0) * emb_weight
    e2t = (emb_weight ** 2).sum(axis=1)[:, None]

    gh, gw = h // _HB, wd // _WB
    nstep = b * gh * gw
    grid = (b, gh, gw)

    def _pidx(i, j, k):
        return (i * gh + j) * gw + k

    oh, zqt, idx3, esum_p, zsum_p, scal_p = pl.pallas_call(
        _vq_body,
        grid=grid,
        in_specs=[
            pl.BlockSpec((1, c, _HB, _WB), lambda i, j, k: (i, 0, j, k)),
            pl.BlockSpec((1, _HB, _WB), lambda i, j, k: (i, j, k)),
            pl.BlockSpec((_KCODES, _EMB), lambda i, j, k: (0, 0)),
            pl.BlockSpec((_KCODES, _EMB), lambda i, j, k: (0, 0)),
            pl.BlockSpec((_KCODES, 1), lambda i, j, k: (0, 0)),
        ],
        out_specs=[
            pl.BlockSpec((1, _HB, _WB, _KCODES), lambda i, j, k: (i, j, k, 0)),
            pl.BlockSpec((1, c, _HB, _WB), lambda i, j, k: (i, 0, j, k)),
            pl.BlockSpec((1, _HB, _WB), lambda i, j, k: (i, j, k)),
            pl.BlockSpec((1, 1, _KCODES), lambda i, j, k: (_pidx(i, j, k), 0, 0)),
            pl.BlockSpec((1, _EMB, _WB), lambda i, j, k: (_pidx(i, j, k), 0, 0)),
            pl.BlockSpec((1, 1, 128), lambda i, j, k: (_pidx(i, j, k), 0, 0)),
        ],
        out_shape=[
            jax.ShapeDtypeStruct((b, h, wd, _KCODES), jnp.float32),
            jax.ShapeDtypeStruct((b, c, h, wd), jnp.float32),
            jax.ShapeDtypeStruct((b, h, wd), jnp.int32),
            jax.ShapeDtypeStruct((nstep, 1, _KCODES), jnp.float32),
            jax.ShapeDtypeStruct((nstep, _EMB, _WB), jnp.float32),
            jax.ShapeDtypeStruct((nstep, 1, 128), jnp.float32),
        ],
        compiler_params=pltpu.CompilerParams(
            dimension_semantics=("parallel", "parallel", "parallel")),
    )(zc, z2, en, emb_weight, e2t)

    min_encodings = oh.reshape(n, _KCODES)
    min_encoding_indices = idx3.reshape(n, 1)
    z_q_st = zqt.reshape(b, c, h, w, dd)
    dminsum = jnp.sum(scal_p[:, 0, 0])
    z2sum = jnp.sum(scal_p[:, 0, 1])
    esum = jnp.sum(esum_p[:, 0, :], axis=0)
    zsum = jnp.sum(zsum_p, axis=(0, 2))
    dsum = (jnp.float32(_KCODES) * z2sum + jnp.float32(n) * jnp.sum(e2t)
            - 2.0 * jnp.sum(zsum * jnp.sum(emb_weight, axis=0)))
    mean_distance = dsum / jnp.float32(n * _KCODES)
    loss = (1.0 + _BETA) * dminsum / jnp.float32(n * c)
    e_mean = esum / jnp.float32(n)
    perplexity = jnp.exp(-jnp.sum(e_mean * jnp.log(e_mean + 1e-10)))
    return (z_q_st, loss, perplexity, min_encodings, min_encoding_indices,
            mean_distance)


# speedup vs baseline: 1.2890x; 1.1184x over previous
import jax
import jax.numpy as jnp
from jax.experimental import pallas as pl
from jax.experimental.pallas import tpu as pltpu

_KCODES = 512
_EMB = 32
_BETA = 0.25
_HB = 8
_WB = 256


def _vq_body(zc_ref, z2_ref, en_ref, e_ref, e2t_ref, oh_ref, zq_ref, idx_ref,
             esum_ref, zsum_ref, scal_ref):
    en = en_ref[...]
    e = e_ref[...]
    e2t = e2t_ref[...]
    esum_acc = jnp.zeros((1, _KCODES), jnp.float32)
    zsum_acc = jnp.zeros((_EMB, _WB), jnp.float32)
    dminsum = jnp.float32(0.0)
    z2sum = jnp.float32(0.0)
    for g in range(_HB):
        zg = zc_ref[0, :, g, :]
        z2g = z2_ref[0, g, :]
        m2t = jax.lax.dot_general(en, zg, (((1,), (0,)), ((), ())),
                                  preferred_element_type=jnp.float32)
        dt = (z2g[None, :] + e2t) + m2t
        dmint = jnp.min(dt, axis=0, keepdims=True)
        citer = jax.lax.broadcasted_iota(jnp.int32, dt.shape, 0)
        idxt = jnp.min(jnp.where(dt == dmint, citer, jnp.int32(_KCODES)),
                       axis=0)
        idx_ref[0, g, :] = idxt
        lane = jax.lax.broadcasted_iota(jnp.int32, (_WB, _KCODES), 1)
        oh = (lane == idxt[:, None]).astype(jnp.float32)
        oh_ref[0, g, :, :] = oh
        zqt = jax.lax.dot_general(e, oh, (((0,), (1,)), ((), ())),
                                  preferred_element_type=jnp.float32)
        zq_ref[0, :, g, :] = zqt
        esum_acc += jnp.sum(oh, axis=0)[None, :]
        zsum_acc += zg
        dminsum += jnp.sum(dmint)
        z2sum += jnp.sum(z2g)
    esum_ref[0] = esum_acc
    zsum_ref[0] = zsum_acc
    scal_ref[0] = (jnp.pad(dminsum.reshape(1, 1), ((0, 0), (0, 127)))
                   + jnp.pad(z2sum.reshape(1, 1), ((0, 0), (1, 126))))


def kernel(z, emb_weight):
    b, c, h, w, dd = z.shape
    n = b * h * w * dd
    wd = w * dd
    zc = z.reshape(b, c, h, wd)
    z2 = (z ** 2).sum(axis=1).reshape(b, h, wd)
    en = jnp.float32(-2.0) * emb_weight
    e2t = (emb_weight ** 2).sum(axis=1)[:, None]

    gh, gw = h // _HB, wd // _WB
    nstep = b * gh * gw
    grid = (b, gh, gw)

    def _pidx(i, j, k):
        return (i * gh + j) * gw + k

    oh, zqt, idx3, esum_p, zsum_p, scal_p = pl.pallas_call(
        _vq_body,
        grid=grid,
        in_specs=[
            pl.BlockSpec((1, c, _HB, _WB), lambda i, j, k: (i, 0, j, k)),
            pl.BlockSpec((1, _HB, _WB), lambda i, j, k: (i, j, k)),
            pl.BlockSpec((_KCODES, _EMB), lambda i, j, k: (0, 0)),
            pl.BlockSpec((_KCODES, _EMB), lambda i, j, k: (0, 0)),
            pl.BlockSpec((_KCODES, 1), lambda i, j, k: (0, 0)),
        ],
        out_specs=[
            pl.BlockSpec((1, _HB, _WB, _KCODES), lambda i, j, k: (i, j, k, 0)),
            pl.BlockSpec((1, c, _HB, _WB), lambda i, j, k: (i, 0, j, k)),
            pl.BlockSpec((1, _HB, _WB), lambda i, j, k: (i, j, k)),
            pl.BlockSpec((1, 1, _KCODES), lambda i, j, k: (_pidx(i, j, k), 0, 0)),
            pl.BlockSpec((1, _EMB, _WB), lambda i, j, k: (_pidx(i, j, k), 0, 0)),
            pl.BlockSpec((1, 1, 128), lambda i, j, k: (_pidx(i, j, k), 0, 0)),
        ],
        out_shape=[
            jax.ShapeDtypeStruct((b, h, wd, _KCODES), jnp.float32),
            jax.ShapeDtypeStruct((b, c, h, wd), jnp.float32),
            jax.ShapeDtypeStruct((b, h, wd), jnp.int32),
            jax.ShapeDtypeStruct((nstep, 1, _KCODES), jnp.float32),
            jax.ShapeDtypeStruct((nstep, _EMB, _WB), jnp.float32),
            jax.ShapeDtypeStruct((nstep, 1, 128), jnp.float32),
        ],
        compiler_params=pltpu.CompilerParams(
            dimension_semantics=("parallel", "parallel", "parallel")),
    )(zc, z2, en, emb_weight, e2t)

    min_encodings = oh.reshape(n, _KCODES)
    min_encoding_indices = idx3.reshape(n, 1)
    z_q_st = zqt.reshape(b, c, h, w, dd)
    dminsum = jnp.sum(scal_p[:, 0, 0])
    z2sum = jnp.sum(scal_p[:, 0, 1])
    esum = jnp.sum(esum_p[:, 0, :], axis=0)
    zsum = jnp.sum(zsum_p, axis=(0, 2))
    dsum = (jnp.float32(_KCODES) * z2sum + jnp.float32(n) * jnp.sum(e2t)
            - 2.0 * jnp.sum(zsum * jnp.sum(emb_weight, axis=0)))
    mean_distance = dsum / jnp.float32(n * _KCODES)
    loss = (1.0 + _BETA) * dminsum / jnp.float32(n * c)
    e_mean = esum / jnp.float32(n)
    perplexity = jnp.exp(-jnp.sum(e_mean * jnp.log(e_mean + 1e-10)))
    return (z_q_st, loss, perplexity, min_encodings, min_encoding_indices,
            mean_distance)


# speedup vs baseline: 1.3082x; 1.0149x over previous
import jax
import jax.numpy as jnp
from jax.experimental import pallas as pl
from jax.experimental.pallas import tpu as pltpu

_KCODES = 512
_EMB = 32
_BETA = 0.25
_HB = 8
_WB = 256


def _vq_body(zc_ref, z2_ref, en_ref, e_ref, e2f_ref, oh_ref, zq_ref, idx_ref,
             esum_ref, zsum_ref, scal_ref):
    en = en_ref[...]
    e = e_ref[...]
    e2f = e2f_ref[...]
    citer = jax.lax.broadcasted_iota(jnp.int32, (_KCODES, _WB), 0)
    lane = jax.lax.broadcasted_iota(jnp.int32, (_WB, _KCODES), 1)
    esum_acc = jnp.zeros((1, _KCODES), jnp.float32)
    zsum_acc = jnp.zeros((_EMB, _WB), jnp.float32)
    dminsum = jnp.float32(0.0)
    z2sum = jnp.float32(0.0)
    for g in range(_HB):
        zg = zc_ref[0, :, g, :]
        z2g = z2_ref[0, g, :]
        m2t = jax.lax.dot_general(en, zg, (((1,), (0,)), ((), ())),
                                  preferred_element_type=jnp.float32)
        dt = (z2g[None, :] + e2f) + m2t
        dmint = jnp.min(dt, axis=0, keepdims=True)
        idxt = jnp.min(jnp.where(dt == dmint, citer, jnp.int32(_KCODES)),
                       axis=0)
        idx_ref[0, g, :] = idxt
        oh = (lane == idxt[:, None]).astype(jnp.float32)
        oh_ref[0, g, :, :] = oh
        zqt = jax.lax.dot_general(e, oh, (((0,), (1,)), ((), ())),
                                  preferred_element_type=jnp.float32)
        zq_ref[0, :, g, :] = zqt
        esum_acc += jnp.sum(oh, axis=0)[None, :]
        zsum_acc += zg
        dminsum += jnp.sum(dmint)
        z2sum += jnp.sum(z2g)
    esum_ref[0] = esum_acc
    zsum_ref[0] = zsum_acc
    scal_ref[0] = (jnp.pad(dminsum.reshape(1, 1), ((0, 0), (0, 127)))
                   + jnp.pad(z2sum.reshape(1, 1), ((0, 0), (1, 126))))


def kernel(z, emb_weight):
    b, c, h, w, dd = z.shape
    n = b * h * w * dd
    wd = w * dd
    zc = z.reshape(b, c, h, wd)
    z2 = (z ** 2).sum(axis=1).reshape(b, h, wd)
    en = jnp.float32(-2.0) * emb_weight
    e2f = jnp.broadcast_to((emb_weight ** 2).sum(axis=1)[:, None],
                           (_KCODES, _WB))

    gh, gw = h // _HB, wd // _WB
    nstep = b * gh * gw
    grid = (b, gh, gw)

    def _pidx(i, j, k):
        return (i * gh + j) * gw + k

    oh, zqt, idx3, esum_p, zsum_p, scal_p = pl.pallas_call(
        _vq_body,
        grid=grid,
        in_specs=[
            pl.BlockSpec((1, c, _HB, _WB), lambda i, j, k: (i, 0, j, k)),
            pl.BlockSpec((1, _HB, _WB), lambda i, j, k: (i, j, k)),
            pl.BlockSpec((_KCODES, _EMB), lambda i, j, k: (0, 0)),
            pl.BlockSpec((_KCODES, _EMB), lambda i, j, k: (0, 0)),
            pl.BlockSpec((_KCODES, _WB), lambda i, j, k: (0, 0)),
        ],
        out_specs=[
            pl.BlockSpec((1, _HB, _WB, _KCODES), lambda i, j, k: (i, j, k, 0)),
            pl.BlockSpec((1, c, _HB, _WB), lambda i, j, k: (i, 0, j, k)),
            pl.BlockSpec((1, _HB, _WB), lambda i, j, k: (i, j, k)),
            pl.BlockSpec((1, 1, _KCODES), lambda i, j, k: (_pidx(i, j, k), 0, 0)),
            pl.BlockSpec((1, _EMB, _WB), lambda i, j, k: (_pidx(i, j, k), 0, 0)),
            pl.BlockSpec((1, 1, 128), lambda i, j, k: (_pidx(i, j, k), 0, 0)),
        ],
        out_shape=[
            jax.ShapeDtypeStruct((b, h, wd, _KCODES), jnp.float32),
            jax.ShapeDtypeStruct((b, c, h, wd), jnp.float32),
            jax.ShapeDtypeStruct((b, h, wd), jnp.int32),
            jax.ShapeDtypeStruct((nstep, 1, _KCODES), jnp.float32),
            jax.ShapeDtypeStruct((nstep, _EMB, _WB), jnp.float32),
            jax.ShapeDtypeStruct((nstep, 1, 128), jnp.float32),
        ],
        compiler_params=pltpu.CompilerParams(
            dimension_semantics=("parallel", "parallel", "parallel")),
    )(zc, z2, en, emb_weight, e2f)

    min_encodings = oh.reshape(n, _KCODES)
    min_encoding_indices = idx3.reshape(n, 1)
    z_q_st = zqt.reshape(b, c, h, w, dd)
    dminsum = jnp.sum(scal_p[:, 0, 0])
    z2sum = jnp.sum(scal_p[:, 0, 1])
    esum = jnp.sum(esum_p[:, 0, :], axis=0)
    zsum = jnp.sum(zsum_p, axis=(0, 2))
    dsum = (jnp.float32(_KCODES) * z2sum + jnp.float32(n) * jnp.sum(e2f[:, 0])
            - 2.0 * jnp.sum(zsum * jnp.sum(emb_weight, axis=0)))
    mean_distance = dsum / jnp.float32(n * _KCODES)
    loss = (1.0 + _BETA) * dminsum / jnp.float32(n * c)
    e_mean = esum / jnp.float32(n)
    perplexity = jnp.exp(-jnp.sum(e_mean * jnp.log(e_mean + 1e-10)))
    return (z_q_st, loss, perplexity, min_encodings, min_encoding_indices,
            mean_distance)
